# baseline (speedup 1.0000x reference)
import functools

import jax
import jax.numpy as jnp
from jax import lax
from jax.experimental import pallas as pl
from jax.experimental.pallas import tpu as pltpu
from jax.experimental.pallas import tpu_sc as plsc

_E = 8
_CAP = 2

_NC = 2
_NS = 16
_NW = _NC * _NS

_GCHUNK = 32
_DSPLIT = 2


def _sc_gather_body(x_hbm, gidx_hbm, xin_hbm, idx_v, rows_v, sem):
    nrows = gidx_hbm.shape[0]
    per_w = nrows // _NW
    nchunk = per_w // _GCHUNK
    wid = lax.axis_index("s") * _NC + lax.axis_index("c")
    base = wid * per_w
    pltpu.sync_copy(gidx_hbm.at[pl.ds(base, per_w)], idx_v)
    for j in range(nchunk):
        idx_c = idx_v.at[pl.ds(j * _GCHUNK, _GCHUNK)]
        pltpu.async_copy(x_hbm.at[idx_c], rows_v, sem).wait()
        pltpu.sync_copy(rows_v, xin_hbm.at[pl.ds(base + j * _GCHUNK, _GCHUNK)])


def _tc_body(idx_ref, g_ref, x_ref, w_ref, *rest):
    out_ref = rest[-1]
    e = pl.program_id(1)
    S = out_ref.shape[1]
    k = idx_ref.shape[2]

    @pl.when(e == 0)
    def _init():
        out_ref[...] = jnp.zeros_like(out_ref)

    xb = x_ref[0].astype(jnp.bfloat16)
    wb = w_ref[0].astype(jnp.bfloat16)
    y = jnp.dot(xb, wb, preferred_element_type=jnp.float32)
    gy = (y * g_ref[0]).astype(jnp.bfloat16)

    iota_s = lax.broadcasted_iota(jnp.int32, (S, k), 0)
    onehot = (iota_s == idx_ref[0]).astype(jnp.bfloat16)
    out_ref[0] = out_ref[0] + jnp.dot(onehot, gy,
                                      preferred_element_type=jnp.float32)


def kernel(x, gate_weight, W_experts):
    B, S, D = x.shape
    E = gate_weight.shape[0]
    k = int(S / E * _CAP)
    nrows = E * k
    DB = D // _DSPLIT

    logits = jnp.einsum('bsd,ed->bse', x, gate_weight)
    affinity = jax.nn.softmax(logits, axis=-1)
    affinity = jnp.transpose(affinity, (0, 2, 1))
    gating, index = jax.lax.top_k(affinity, k)
    index = index.astype(jnp.int32)

    mesh = plsc.VectorSubcoreMesh(core_axis_name="c", subcore_axis_name="s")
    gather = pl.kernel(
        _sc_gather_body,
        out_type=jax.ShapeDtypeStruct((nrows, D), jnp.float32),
        mesh=mesh,
        scratch_types=[
            pltpu.VMEM((nrows // _NW,), jnp.int32),
            pltpu.VMEM((_GCHUNK, D), jnp.float32),
            pltpu.SemaphoreType.DMA,
        ],
    )
    x_in = [gather(x[b], index[b].reshape(nrows)) for b in range(B)]

    out = None
    for b in range(B):
        in_specs = [
            pl.BlockSpec((1, 1, k), lambda d, e: (e, 0, 0)),
            pl.BlockSpec((1, k, 1), lambda d, e: (e, 0, 0)),
            pl.BlockSpec((1, k, D), lambda d, e: (e, 0, 0)),
            pl.BlockSpec((1, D, DB), lambda d, e: (e, 0, d)),
        ]
        args = [index[b].reshape(E, 1, k), gating[b].reshape(E, k, 1),
                x_in[b].reshape(E, k, D), W_experts]
        aliases = {}
        if out is not None:
            in_specs.append(pl.BlockSpec(memory_space=pl.ANY))
            args.append(out)
            aliases = {4: 0}
        out = pl.pallas_call(
            _tc_body,
            grid=(_DSPLIT, E),
            in_specs=in_specs,
            out_specs=pl.BlockSpec((1, S, DB),
                                   functools.partial(
                                       lambda bb, d, e: (bb, 0, d), b)),
            out_shape=jax.ShapeDtypeStruct((B, S, D), jnp.float32),
            input_output_aliases=aliases,
        )(*args)
    return out


# speedup vs baseline: 1.0759x; 1.0759x over previous
import jax
import jax.numpy as jnp
from jax import lax
from jax.experimental import pallas as pl
from jax.experimental.pallas import tpu as pltpu
from jax.experimental.pallas import tpu_sc as plsc

_E = 8
_CAP = 2

_NC = 2
_NS = 16
_NW = _NC * _NS

_GCHUNK = 16
_DSPLIT = 2


def _sc_gather_body(x_hbm, gidx_hbm, xin_hbm, idx_v, rows0_v, rows1_v,
                    sem0, sem1):
    nrows = gidx_hbm.shape[0]
    per_w = nrows // _NW
    nchunk = per_w // _GCHUNK
    wid = lax.axis_index("s") * _NC + lax.axis_index("c")
    base = wid * per_w
    pltpu.sync_copy(gidx_hbm.at[pl.ds(base, per_w)], idx_v)
    bufs = (rows0_v, rows1_v)
    sems = (sem0, sem1)

    def start(j):
        idx_c = idx_v.at[pl.ds(j * _GCHUNK, _GCHUNK)]
        return pltpu.async_copy(x_hbm.at[idx_c], bufs[j % 2], sems[j % 2])

    h = start(0)
    for j in range(nchunk):
        h.wait()
        if j + 1 < nchunk:
            h = start(j + 1)
        pltpu.sync_copy(bufs[j % 2],
                        xin_hbm.at[pl.ds(base + j * _GCHUNK, _GCHUNK)])


def _tc_body(idx_ref, g_ref, x_ref, w_ref, out_ref):
    e = pl.program_id(2)
    S = out_ref.shape[1]
    k = idx_ref.shape[2]

    @pl.when(e == 0)
    def _init():
        out_ref[...] = jnp.zeros_like(out_ref)

    xb = x_ref[0, 0].astype(jnp.bfloat16)
    wb = w_ref[0].astype(jnp.bfloat16)
    y = jnp.dot(xb, wb, preferred_element_type=jnp.float32)
    gy = (y * g_ref[0]).astype(jnp.bfloat16)

    iota_s = lax.broadcasted_iota(jnp.int32, (S, k), 0)
    onehot = (iota_s == idx_ref[0]).astype(jnp.bfloat16)
    out_ref[0] = out_ref[0] + jnp.dot(onehot, gy,
                                      preferred_element_type=jnp.float32)


def kernel(x, gate_weight, W_experts):
    B, S, D = x.shape
    E = gate_weight.shape[0]
    k = int(S / E * _CAP)
    nrows = B * E * k
    DB = D // _DSPLIT

    logits = jnp.einsum('bsd,ed->bse', x, gate_weight)
    affinity = jax.nn.softmax(logits, axis=-1)
    affinity = jnp.transpose(affinity, (0, 2, 1))
    gating, index = jax.lax.top_k(affinity, k)
    index = index.astype(jnp.int32)

    gidx = (jnp.arange(B, dtype=jnp.int32)[:, None, None] * S
            + index).reshape(nrows)
    idx_row = index.reshape(B * E, 1, k)
    g_col = gating.reshape(B * E, k, 1)
    x_flat = x.reshape(B * S, D)

    mesh = plsc.VectorSubcoreMesh(core_axis_name="c", subcore_axis_name="s")
    gather = pl.kernel(
        _sc_gather_body,
        out_type=jax.ShapeDtypeStruct((nrows, D), jnp.float32),
        mesh=mesh,
        scratch_types=[
            pltpu.VMEM((nrows // _NW,), jnp.int32),
            pltpu.VMEM((_GCHUNK, D), jnp.float32),
            pltpu.VMEM((_GCHUNK, D), jnp.float32),
            pltpu.SemaphoreType.DMA,
            pltpu.SemaphoreType.DMA,
        ],
    )
    x_in = gather(x_flat, gidx)

    out = pl.pallas_call(
        _tc_body,
        grid=(B, _DSPLIT, E),
        in_specs=[
            pl.BlockSpec((1, 1, k), lambda b, d, e: (b * _E + e, 0, 0)),
            pl.BlockSpec((1, k, 1), lambda b, d, e: (b * _E + e, 0, 0)),
            pl.BlockSpec((1, 1, k, D), lambda b, d, e: (b, e, 0, 0)),
            pl.BlockSpec((1, D, DB), lambda b, d, e: (e, 0, d)),
        ],
        out_specs=pl.BlockSpec((1, S, DB), lambda b, d, e: (b, 0, d)),
        out_shape=jax.ShapeDtypeStruct((B, S, D), jnp.float32),
    )(idx_row, g_col, x_in.reshape(B, E, k, D), W_experts)
    return out
